# baseline (speedup 1.0000x reference)
import functools

import jax
import jax.numpy as jnp
from jax import lax
from jax.experimental import pallas as pl
from jax.experimental.pallas import tpu as pltpu
from jax.experimental.pallas import tpu_sc as plsc

N = 10000
NPAD = 10240
E = 320000
CHUNK = 128
NW = 32
RPW = -(-E // (NW * CHUNK))
EPW = RPW * CHUNK
EPAD = NW * EPW
GROUPS = EPW // 16


@functools.lru_cache(maxsize=None)
def _deg_program():
    mesh = plsc.VectorSubcoreMesh(core_axis_name="c", subcore_axis_name="s")

    @functools.partial(
        pl.kernel,
        out_type=jax.ShapeDtypeStruct((NW, NPAD), jnp.float32),
        mesh=mesh,
        compiler_params=pltpu.CompilerParams(needs_layout_passes=False),
        scratch_types=[
            pltpu.VMEM((NPAD,), jnp.float32),
            pltpu.VMEM((EPW,), jnp.int32),
            pltpu.SemaphoreType.DMA,
        ],
    )
    def k(dst_hbm, out_hbm, acc_v, dst_v, sem):
        c = lax.axis_index("c")
        s = lax.axis_index("s")
        wid = s * 2 + c
        cp = pltpu.async_copy(dst_hbm.at[wid], dst_v, sem)

        zero16 = jnp.zeros((16,), jnp.float32)

        @plsc.parallel_loop(0, NPAD // 16, unroll=8)
        def zbody(zi):
            acc_v[pl.ds(pl.multiple_of(zi * 16, 16), 16)] = zero16

        cp.wait()
        one16 = jnp.ones((16,), jnp.float32)

        @plsc.parallel_loop(0, GROUPS, unroll=16)
        def body(gi):
            off = pl.multiple_of(gi * 16, 16)
            dst16 = dst_v[pl.ds(off, 16)]
            plsc.addupdate_scatter(acc_v, [dst16], one16)

        pltpu.sync_copy(acc_v, out_hbm.at[wid])

    return k


@functools.lru_cache(maxsize=None)
def _seg_sum_program(f):
    mesh = plsc.VectorSubcoreMesh(core_axis_name="c", subcore_axis_name="s")

    @functools.partial(
        pl.kernel,
        out_type=jax.ShapeDtypeStruct((NW, NPAD * f), jnp.float32),
        mesh=mesh,
        compiler_params=pltpu.CompilerParams(needs_layout_passes=False),
        scratch_types=[
            pltpu.VMEM((NPAD * f,), jnp.float32),
            pltpu.VMEM((NPAD * f,), jnp.float32),
            pltpu.VMEM((EPW,), jnp.int32),
            pltpu.VMEM((EPW,), jnp.int32),
            pltpu.SemaphoreType.DMA,
        ],
    )
    def k(g_hbm, src_hbm, dst_hbm, out_hbm, g_v, acc_v, src_v, dst_v, sem):
        c = lax.axis_index("c")
        s = lax.axis_index("s")
        wid = s * 2 + c
        cp_g = pltpu.async_copy(g_hbm, g_v, sem)
        cp_s = pltpu.async_copy(src_hbm.at[wid], src_v, sem)
        cp_d = pltpu.async_copy(dst_hbm.at[wid], dst_v, sem)

        zero16 = jnp.zeros((16,), jnp.float32)

        @plsc.parallel_loop(0, (NPAD * f) // 16, unroll=8)
        def zbody(zi):
            acc_v[pl.ds(pl.multiple_of(zi * 16, 16), 16)] = zero16

        cp_g.wait()
        cp_s.wait()
        cp_d.wait()

        @plsc.parallel_loop(0, GROUPS, unroll=16)
        def body(gi):
            off = pl.multiple_of(gi * 16, 16)
            src16 = src_v[pl.ds(off, 16)]
            dst16 = dst_v[pl.ds(off, 16)]
            for k in range(f):
                v = plsc.load_gather(g_v, [src16 + (k * NPAD)])
                plsc.addupdate_scatter(acc_v, [dst16 + (k * NPAD)], v)

        pltpu.sync_copy(acc_v, out_hbm.at[wid])

    return k


def _seg_sum(f, g_flat, srcw, dstw):
    return _seg_sum_program(f)(g_flat, srcw, dstw)


def _reduce_parts(sp_ref):
    acc = sp_ref[0]
    for i in range(1, NW):
        acc = acc + sp_ref[i]
    return acc


def _tc1_body(degp_ref, xt_ref, w1t_ref, dinv_ref, t_ref, g_ref):
    deg = _reduce_parts(degp_ref) + 1.0
    dinv = lax.rsqrt(deg)
    t = jnp.dot(w1t_ref[...], xt_ref[...], preferred_element_type=jnp.float32)
    dinv_ref[...] = dinv
    t_ref[...] = t
    g_ref[...] = dinv * t


def _tc_mid_body(sp_ref, dinv_ref, t_ref, b_ref, wt_ref, t2_ref, g2_ref):
    dinv = dinv_ref[...]
    s = _reduce_parts(sp_ref)
    h = jnp.tanh(dinv * s + dinv * dinv * t_ref[...] + b_ref[...])
    t2 = jnp.dot(wt_ref[...], h, preferred_element_type=jnp.float32)
    t2_ref[...] = t2
    g2_ref[...] = dinv * t2


def _tc4_body(sp_ref, dinv_ref, t_ref, b_ref, wlt_ref, blt_ref, out_ref,
              h_ref):
    dinv = dinv_ref[...]
    s = _reduce_parts(sp_ref)
    h = dinv * s + dinv * dinv * t_ref[...] + b_ref[...]
    h_ref[...] = h
    out_ref[...] = (jnp.dot(wlt_ref[...], h, preferred_element_type=jnp.float32)
                    + blt_ref[...])


def kernel(x, edge_index, W1, b1, W2, b2, W3, b3, Wl, bl):
    src = edge_index[0]
    dst = edge_index[1]
    npad_e = EPAD - E
    pad_idx = N + (jnp.arange(npad_e, dtype=jnp.int32) % (NPAD - N))
    src_p = jnp.concatenate([src, pad_idx])
    dst_p = jnp.concatenate([dst, pad_idx])
    srcw = src_p.reshape(RPW, NW, CHUNK).transpose(1, 0, 2).reshape(NW, EPW)
    dstw = dst_p.reshape(RPW, NW, CHUNK).transpose(1, 0, 2).reshape(NW, EPW)

    xt = jnp.pad(x, ((0, NPAD - N), (0, 0))).T

    sds = jax.ShapeDtypeStruct

    degp = _deg_program()(dstw)

    dinv, t1, g1 = pl.pallas_call(
        _tc1_body,
        out_shape=[sds((1, NPAD), jnp.float32), sds((4, NPAD), jnp.float32),
                   sds((4, NPAD), jnp.float32)],
    )(degp.reshape(NW, 1, NPAD), xt, W1.T)

    s1p = _seg_sum(4, g1.reshape(-1), srcw, dstw)
    t2, g2 = pl.pallas_call(
        _tc_mid_body,
        out_shape=[sds((4, NPAD), jnp.float32), sds((4, NPAD), jnp.float32)],
    )(s1p.reshape(NW, 4, NPAD), dinv, t1, b1.reshape(-1, 1), W2.T)

    s2p = _seg_sum(4, g2.reshape(-1), srcw, dstw)
    t3, g3 = pl.pallas_call(
        _tc_mid_body,
        out_shape=[sds((2, NPAD), jnp.float32), sds((2, NPAD), jnp.float32)],
    )(s2p.reshape(NW, 4, NPAD), dinv, t2, b2.reshape(-1, 1), W3.T)

    s3p = _seg_sum(2, g3.reshape(-1), srcw, dstw)
    out_t, h_t = pl.pallas_call(
        _tc4_body,
        out_shape=[sds((4, NPAD), jnp.float32), sds((2, NPAD), jnp.float32)],
    )(s3p.reshape(NW, 2, NPAD), dinv, t3, b3.reshape(-1, 1), Wl.T,
      bl.reshape(-1, 1))

    return (out_t.T[:N], h_t.T[:N])


# speedup vs baseline: 1.0211x; 1.0211x over previous
import functools

import jax
import jax.numpy as jnp
from jax import lax
from jax.experimental import pallas as pl
from jax.experimental.pallas import tpu as pltpu
from jax.experimental.pallas import tpu_sc as plsc

N = 10000
NPAD = 10240
E = 320000
CHUNK = 128
NW = 32
RPW = -(-E // (NW * CHUNK))
EPW = RPW * CHUNK
EPAD = NW * EPW
GROUPS = EPW // 16


@functools.lru_cache(maxsize=None)
def _deg_program():
    mesh = plsc.VectorSubcoreMesh(core_axis_name="c", subcore_axis_name="s")

    @functools.partial(
        pl.kernel,
        out_type=jax.ShapeDtypeStruct((NW, NPAD), jnp.float32),
        mesh=mesh,
        compiler_params=pltpu.CompilerParams(needs_layout_passes=False),
        scratch_types=[
            pltpu.VMEM((NPAD,), jnp.float32),
            pltpu.VMEM((EPW,), jnp.int32),
            pltpu.SemaphoreType.DMA,
        ],
    )
    def k(dst_hbm, out_hbm, acc_v, dst_v, sem):
        c = lax.axis_index("c")
        s = lax.axis_index("s")
        wid = s * 2 + c
        cp = pltpu.async_copy(dst_hbm.at[wid], dst_v, sem)

        zero16 = jnp.zeros((16,), jnp.float32)

        @plsc.parallel_loop(0, NPAD // 16, unroll=8)
        def zbody(zi):
            acc_v[pl.ds(pl.multiple_of(zi * 16, 16), 16)] = zero16

        cp.wait()
        one16 = jnp.ones((16,), jnp.float32)

        @plsc.parallel_loop(0, GROUPS, unroll=8)
        def body(gi):
            off = pl.multiple_of(gi * 16, 16)
            dst16 = dst_v[pl.ds(off, 16)]
            plsc.addupdate_scatter(acc_v, [dst16], one16)

        pltpu.sync_copy(acc_v, out_hbm.at[wid])

    return k


@functools.lru_cache(maxsize=None)
def _seg_sum_program(f):
    mesh = plsc.VectorSubcoreMesh(core_axis_name="c", subcore_axis_name="s")

    @functools.partial(
        pl.kernel,
        out_type=jax.ShapeDtypeStruct((NW, NPAD * f), jnp.float32),
        mesh=mesh,
        compiler_params=pltpu.CompilerParams(needs_layout_passes=False),
        scratch_types=[
            pltpu.VMEM((NPAD * f,), jnp.float32),
            pltpu.VMEM((NPAD * f,), jnp.float32),
            pltpu.VMEM((EPW,), jnp.int32),
            pltpu.VMEM((EPW,), jnp.int32),
            pltpu.SemaphoreType.DMA,
        ],
    )
    def k(g_hbm, src_hbm, dst_hbm, out_hbm, g_v, acc_v, src_v, dst_v, sem):
        c = lax.axis_index("c")
        s = lax.axis_index("s")
        wid = s * 2 + c
        cp_g = pltpu.async_copy(g_hbm, g_v, sem)
        cp_s = pltpu.async_copy(src_hbm.at[wid], src_v, sem)
        cp_d = pltpu.async_copy(dst_hbm.at[wid], dst_v, sem)

        zero16 = jnp.zeros((16,), jnp.float32)

        @plsc.parallel_loop(0, (NPAD * f) // 16, unroll=8)
        def zbody(zi):
            acc_v[pl.ds(pl.multiple_of(zi * 16, 16), 16)] = zero16

        cp_g.wait()
        cp_s.wait()
        cp_d.wait()

        @plsc.parallel_loop(0, GROUPS, unroll=8)
        def body(gi):
            off = pl.multiple_of(gi * 16, 16)
            src16 = src_v[pl.ds(off, 16)]
            dst16 = dst_v[pl.ds(off, 16)]
            for k in range(f):
                v = plsc.load_gather(g_v, [src16 + (k * NPAD)])
                plsc.addupdate_scatter(acc_v, [dst16 + (k * NPAD)], v)

        pltpu.sync_copy(acc_v, out_hbm.at[wid])

    return k


def _seg_sum(f, g_flat, srcw, dstw):
    return _seg_sum_program(f)(g_flat, srcw, dstw)


def _reduce_parts(sp_ref):
    acc = sp_ref[0]
    for i in range(1, NW):
        acc = acc + sp_ref[i]
    return acc


def _tc1_body(degp_ref, xt_ref, w1t_ref, dinv_ref, t_ref, g_ref):
    deg = _reduce_parts(degp_ref) + 1.0
    dinv = lax.rsqrt(deg)
    t = jnp.dot(w1t_ref[...], xt_ref[...], preferred_element_type=jnp.float32)
    dinv_ref[...] = dinv
    t_ref[...] = t
    g_ref[...] = dinv * t


def _tc_mid_body(sp_ref, dinv_ref, t_ref, b_ref, wt_ref, t2_ref, g2_ref):
    dinv = dinv_ref[...]
    s = _reduce_parts(sp_ref)
    h = jnp.tanh(dinv * s + dinv * dinv * t_ref[...] + b_ref[...])
    t2 = jnp.dot(wt_ref[...], h, preferred_element_type=jnp.float32)
    t2_ref[...] = t2
    g2_ref[...] = dinv * t2


def _tc4_body(sp_ref, dinv_ref, t_ref, b_ref, wlt_ref, blt_ref, out_ref,
              h_ref):
    dinv = dinv_ref[...]
    s = _reduce_parts(sp_ref)
    h = dinv * s + dinv * dinv * t_ref[...] + b_ref[...]
    h_ref[...] = h
    out_ref[...] = (jnp.dot(wlt_ref[...], h, preferred_element_type=jnp.float32)
                    + blt_ref[...])


def kernel(x, edge_index, W1, b1, W2, b2, W3, b3, Wl, bl):
    src = edge_index[0]
    dst = edge_index[1]
    npad_e = EPAD - E
    pad_idx = N + (jnp.arange(npad_e, dtype=jnp.int32) % (NPAD - N))
    src_p = jnp.concatenate([src, pad_idx])
    dst_p = jnp.concatenate([dst, pad_idx])
    srcw = src_p.reshape(RPW, NW, CHUNK).transpose(1, 0, 2).reshape(NW, EPW)
    dstw = dst_p.reshape(RPW, NW, CHUNK).transpose(1, 0, 2).reshape(NW, EPW)

    xt = jnp.pad(x, ((0, NPAD - N), (0, 0))).T

    sds = jax.ShapeDtypeStruct

    degp = _deg_program()(dstw)

    dinv, t1, g1 = pl.pallas_call(
        _tc1_body,
        out_shape=[sds((1, NPAD), jnp.float32), sds((4, NPAD), jnp.float32),
                   sds((4, NPAD), jnp.float32)],
    )(degp.reshape(NW, 1, NPAD), xt, W1.T)

    s1p = _seg_sum(4, g1.reshape(-1), srcw, dstw)
    t2, g2 = pl.pallas_call(
        _tc_mid_body,
        out_shape=[sds((4, NPAD), jnp.float32), sds((4, NPAD), jnp.float32)],
    )(s1p.reshape(NW, 4, NPAD), dinv, t1, b1.reshape(-1, 1), W2.T)

    s2p = _seg_sum(4, g2.reshape(-1), srcw, dstw)
    t3, g3 = pl.pallas_call(
        _tc_mid_body,
        out_shape=[sds((2, NPAD), jnp.float32), sds((2, NPAD), jnp.float32)],
    )(s2p.reshape(NW, 4, NPAD), dinv, t2, b2.reshape(-1, 1), W3.T)

    s3p = _seg_sum(2, g3.reshape(-1), srcw, dstw)
    out_t, h_t = pl.pallas_call(
        _tc4_body,
        out_shape=[sds((4, NPAD), jnp.float32), sds((2, NPAD), jnp.float32)],
    )(s3p.reshape(NW, 2, NPAD), dinv, t3, b3.reshape(-1, 1), Wl.T,
      bl.reshape(-1, 1))

    return (out_t.T[:N], h_t.T[:N])
